# baseline (speedup 1.0000x reference)
import functools

import jax
import jax.numpy as jnp
from jax import lax
from jax.experimental import pallas as pl
from jax.experimental.pallas import tpu as pltpu
from jax.experimental.pallas import tpu_sc as plsc

_CHUNK = 64
_NBUF = 3


def kernel(indices, logits):
    B, = indices.shape
    V, D = logits.shape
    info = plsc.get_sparse_core_info()
    NC, NS, L = info.num_cores, info.num_subcores, info.num_lanes
    NW = NC * NS
    b_per_w = B // NW
    n_chunks = b_per_w // _CHUNK
    mesh = plsc.VectorSubcoreMesh(core_axis_name="c", subcore_axis_name="s")

    @functools.partial(
        pl.kernel,
        mesh=mesh,
        out_type=jax.ShapeDtypeStruct((B, D), jnp.float32),
        scratch_types=[
            pltpu.VMEM((b_per_w,), jnp.int32),
        ] + [pltpu.VMEM((_CHUNK, D), jnp.float32)] * _NBUF
          + [pltpu.SemaphoreType.DMA] * (2 * _NBUF),
    )
    def _run(idx_hbm, table_hbm, out_hbm, idx_v, *rest):
        bufs = rest[:_NBUF]
        gsems = rest[_NBUF:2 * _NBUF]
        wsems = rest[2 * _NBUF:3 * _NBUF]
        wid = lax.axis_index("s") * NC + lax.axis_index("c")
        base = wid * b_per_w
        pltpu.sync_copy(idx_hbm.at[pl.ds(base, b_per_w)], idx_v)

        def start_gather(g):
            s = g % _NBUF
            return pltpu.async_copy(
                table_hbm.at[idx_v.at[pl.ds(g * _CHUNK, _CHUNK)]],
                bufs[s], gsems[s])

        gcopies = [None] * n_chunks
        wcopies = [None] * n_chunks
        for g in range(min(2, n_chunks)):
            gcopies[g] = start_gather(g)
        for g in range(n_chunks):
            s = g % _NBUF
            if g + 2 < n_chunks:
                if g >= 1:
                    wcopies[g - 1].wait()
                gcopies[g + 2] = start_gather(g + 2)
            gcopies[g].wait()
            buf = bufs[s]

            @plsc.parallel_loop(0, _CHUNK, unroll=2)
            def _sigmoid_rows(r):
                for c in range(D // L):
                    x = buf[r, pl.ds(c * L, L)]
                    buf[r, pl.ds(c * L, L)] = 1.0 / (1.0 + jnp.exp(-x))

            wcopies[g] = pltpu.async_copy(
                buf, out_hbm.at[pl.ds(base + g * _CHUNK, _CHUNK)], wsems[s])
        for g in range(max(0, n_chunks - 3), n_chunks):
            wcopies[g].wait()

    return _run(indices, logits)


# speedup vs baseline: 1.0359x; 1.0359x over previous
import functools

import jax
import jax.numpy as jnp
from jax import lax
from jax.experimental import pallas as pl
from jax.experimental.pallas import tpu as pltpu
from jax.experimental.pallas import tpu_sc as plsc

_CHUNK = 128
_NBUF = 3


def kernel(indices, logits):
    B, = indices.shape
    V, D = logits.shape
    info = plsc.get_sparse_core_info()
    NC, NS, L = info.num_cores, info.num_subcores, info.num_lanes
    NW = NC * NS
    b_per_w = B // NW
    n_chunks = b_per_w // _CHUNK
    mesh = plsc.VectorSubcoreMesh(core_axis_name="c", subcore_axis_name="s")

    @functools.partial(
        pl.kernel,
        mesh=mesh,
        out_type=jax.ShapeDtypeStruct((B, D), jnp.float32),
        scratch_types=[
            pltpu.VMEM((b_per_w,), jnp.int32),
        ] + [pltpu.VMEM((_CHUNK, D), jnp.float32)] * _NBUF
          + [pltpu.SemaphoreType.DMA] * (2 * _NBUF),
    )
    def _run(idx_hbm, table_hbm, out_hbm, idx_v, *rest):
        bufs = rest[:_NBUF]
        gsems = rest[_NBUF:2 * _NBUF]
        wsems = rest[2 * _NBUF:3 * _NBUF]
        wid = lax.axis_index("s") * NC + lax.axis_index("c")
        base = wid * b_per_w
        pltpu.sync_copy(idx_hbm.at[pl.ds(base, b_per_w)], idx_v)

        def start_gather(g):
            s = g % _NBUF
            return pltpu.async_copy(
                table_hbm.at[idx_v.at[pl.ds(g * _CHUNK, _CHUNK)]],
                bufs[s], gsems[s])

        gcopies = [None] * n_chunks
        wcopies = [None] * n_chunks
        for g in range(min(2, n_chunks)):
            gcopies[g] = start_gather(g)
        for g in range(n_chunks):
            s = g % _NBUF
            if g + 2 < n_chunks:
                if g >= 1:
                    wcopies[g - 1].wait()
                gcopies[g + 2] = start_gather(g + 2)
            gcopies[g].wait()
            buf = bufs[s]

            @plsc.parallel_loop(0, _CHUNK, unroll=4)
            def _sigmoid_rows(r):
                for c in range(D // L):
                    x = buf[r, pl.ds(c * L, L)]
                    buf[r, pl.ds(c * L, L)] = 1.0 / (1.0 + jnp.exp(-x))

            wcopies[g] = pltpu.async_copy(
                buf, out_hbm.at[pl.ds(base + g * _CHUNK, _CHUNK)], wsems[s])
        for g in range(max(0, n_chunks - 3), n_chunks):
            wcopies[g].wait()

    return _run(indices, logits)


# speedup vs baseline: 1.0373x; 1.0013x over previous
import functools

import jax
import jax.numpy as jnp
from jax import lax
from jax.experimental import pallas as pl
from jax.experimental.pallas import tpu as pltpu
from jax.experimental.pallas import tpu_sc as plsc

_CHUNK = 128
_NBUF = 4


def kernel(indices, logits):
    B, = indices.shape
    V, D = logits.shape
    info = plsc.get_sparse_core_info()
    NC, NS, L = info.num_cores, info.num_subcores, info.num_lanes
    NW = NC * NS
    b_per_w = B // NW
    n_chunks = b_per_w // _CHUNK
    mesh = plsc.VectorSubcoreMesh(core_axis_name="c", subcore_axis_name="s")

    @functools.partial(
        pl.kernel,
        mesh=mesh,
        out_type=jax.ShapeDtypeStruct((B, D), jnp.float32),
        scratch_types=[
            pltpu.VMEM((b_per_w,), jnp.int32),
        ] + [pltpu.VMEM((_CHUNK, D), jnp.float32)] * _NBUF
          + [pltpu.SemaphoreType.DMA] * (2 * _NBUF),
    )
    def _run(idx_hbm, table_hbm, out_hbm, idx_v, *rest):
        bufs = rest[:_NBUF]
        gsems = rest[_NBUF:2 * _NBUF]
        wsems = rest[2 * _NBUF:3 * _NBUF]
        wid = lax.axis_index("s") * NC + lax.axis_index("c")
        base = wid * b_per_w
        pltpu.sync_copy(idx_hbm.at[pl.ds(base, b_per_w)], idx_v)

        def start_gather(g):
            s = g % _NBUF
            return pltpu.async_copy(
                table_hbm.at[idx_v.at[pl.ds(g * _CHUNK, _CHUNK)]],
                bufs[s], gsems[s])

        gcopies = [None] * n_chunks
        wcopies = [None] * n_chunks
        for g in range(n_chunks):
            gcopies[g] = start_gather(g)
        for g in range(n_chunks):
            s = g % _NBUF
            gcopies[g].wait()
            buf = bufs[s]

            @plsc.parallel_loop(0, _CHUNK, unroll=4)
            def _sigmoid_rows(r):
                for c in range(D // L):
                    x = buf[r, pl.ds(c * L, L)]
                    buf[r, pl.ds(c * L, L)] = 1.0 / (1.0 + jnp.exp(-x))

            wcopies[g] = pltpu.async_copy(
                buf, out_hbm.at[pl.ds(base + g * _CHUNK, _CHUNK)], wsems[s])
        for g in range(n_chunks):
            wcopies[g].wait()

    return _run(indices, logits)
